# baseline (speedup 1.0000x reference)
import functools

import jax
import jax.numpy as jnp
from jax import lax
from jax.experimental import pallas as pl
from jax.experimental.pallas import tpu as pltpu
from jax.experimental.pallas import tpu_sc as plsc

_NUM_EMB = 1024
_DIM = 64
_COMMIT = 0.25
_B = 16
_F = 576
_TOKENS = _B * _F
_BLK = 4608
_G = _TOKENS // _BLK


def _tc_distance_argmin(z_ref, e_ref, idx_ref, loss_ref, e2_ref, iota_ref):
    i = pl.program_id(0)
    z = z_ref[...]
    e = e_ref[...]

    @pl.when(i == 0)
    def _prep():
        e2_ref[...] = jnp.sum(e * e, axis=1)
        iota_ref[...] = lax.broadcasted_iota(
            jnp.int32, (1, _NUM_EMB), 1).astype(jnp.float32)
        loss_ref[0, 0] = 0.0

    m = lax.dot_general(z * (-2.0), e, (((1,), (1,)), ((), ())),
                        preferred_element_type=jnp.float32)
    z2 = jnp.sum(z * z, axis=1, keepdims=True)
    d = (z2 + m) + e2_ref[...][None, :]
    mind = jnp.min(d, axis=1, keepdims=True)
    idxf = jnp.min(jnp.where(d == mind, iota_ref[...], float(_NUM_EMB)),
                   axis=1)
    idx_ref[pl.ds(i * _BLK, _BLK)] = idxf.astype(jnp.int32)
    loss_ref[0, 0] += jnp.sum(mind)

    @pl.when(i == _G - 1)
    def _fin():
        loss_ref[0, 0] *= _COMMIT / (_TOKENS * _DIM)


_tc_call = pl.pallas_call(
    _tc_distance_argmin,
    grid=(_G,),
    in_specs=[
        pl.BlockSpec((_BLK, _DIM), lambda i: (i, 0)),
        pl.BlockSpec((_NUM_EMB, _DIM), lambda i: (0, 0)),
    ],
    out_specs=[
        pl.BlockSpec((_TOKENS,), lambda i: (0,)),
        pl.BlockSpec((1, 1), lambda i: (0, 0), memory_space=pltpu.SMEM),
    ],
    out_shape=[
        jax.ShapeDtypeStruct((_TOKENS,), jnp.int32),
        jax.ShapeDtypeStruct((1, 1), jnp.float32),
    ],
    scratch_shapes=[pltpu.VMEM((_NUM_EMB,), jnp.float32),
                    pltpu.VMEM((1, _NUM_EMB), jnp.float32)],
)


@functools.lru_cache(maxsize=None)
def _make_sc_gather(nc, ns):
    nw = nc * ns
    b_per_w = _TOKENS // nw
    mesh = plsc.VectorSubcoreMesh(core_axis_name="c", subcore_axis_name="s",
                                  num_cores=nc, num_subcores=ns)

    @functools.partial(
        pl.kernel,
        mesh=mesh,
        compiler_params=pltpu.CompilerParams(use_tc_tiling_on_sc=False),
        out_type=jax.ShapeDtypeStruct((_B, _F, _DIM), jnp.float32),
        scratch_types=[
            pltpu.VMEM((b_per_w,), jnp.int32),
            pltpu.VMEM((b_per_w, _DIM), jnp.float32),
            pltpu.SemaphoreType.DMA,
        ],
    )
    def _gather(e_hbm, idx_hbm, out_hbm, idx_v, rows_v, sem):
        wid = lax.axis_index("s") * nc + lax.axis_index("c")
        row = (wid * b_per_w) // _F
        col = (wid * b_per_w) % _F
        pltpu.sync_copy(idx_hbm.at[pl.ds(wid * b_per_w, b_per_w)], idx_v)
        pltpu.async_copy(e_hbm.at[idx_v], rows_v, sem).wait()
        pltpu.sync_copy(rows_v, out_hbm.at[row, pl.ds(col, b_per_w), :])

    return _gather


def kernel(z, embedding):
    b, f, dim = z.shape
    idx, loss = _tc_call(z.reshape(-1, dim), embedding)
    info = plsc.get_sparse_core_info()
    zq = _make_sc_gather(1, info.num_subcores)(embedding, idx)
    return zq, idx.reshape(b, f), loss[0, 0]


# speedup vs baseline: 1.0029x; 1.0029x over previous
import functools

import jax
import jax.numpy as jnp
from jax import lax
from jax.experimental import pallas as pl
from jax.experimental.pallas import tpu as pltpu
from jax.experimental.pallas import tpu_sc as plsc

_NUM_EMB = 1024
_DIM = 64
_COMMIT = 0.25
_B = 16
_F = 576
_TOKENS = _B * _F
_BLK = 2304
_G = _TOKENS // _BLK


def _tc_distance_argmin(z_ref, e_ref, idx_ref, loss_ref, e2_ref, iota_ref):
    i = pl.program_id(0)
    z = z_ref[...]
    e = e_ref[...]

    @pl.when(i == 0)
    def _prep():
        e2_ref[...] = jnp.sum(e * e, axis=1)
        iota_ref[...] = lax.broadcasted_iota(
            jnp.int32, (1, _NUM_EMB), 1).astype(jnp.float32)
        loss_ref[0, 0] = 0.0

    m = lax.dot_general(z * (-2.0), e, (((1,), (1,)), ((), ())),
                        preferred_element_type=jnp.float32)
    z2 = jnp.sum(z * z, axis=1, keepdims=True)
    d = (z2 + m) + e2_ref[...][None, :]
    mind = jnp.min(d, axis=1, keepdims=True)
    idxf = jnp.min(jnp.where(d == mind, iota_ref[...], float(_NUM_EMB)),
                   axis=1)
    idx_ref[pl.ds(i * _BLK, _BLK)] = idxf.astype(jnp.int32)
    loss_ref[0, 0] += jnp.sum(mind)

    @pl.when(i == _G - 1)
    def _fin():
        loss_ref[0, 0] *= _COMMIT / (_TOKENS * _DIM)


_tc_call = pl.pallas_call(
    _tc_distance_argmin,
    grid=(_G,),
    compiler_params=pltpu.CompilerParams(allow_input_fusion=[True, True]),
    in_specs=[
        pl.BlockSpec((_BLK, _DIM), lambda i: (i, 0)),
        pl.BlockSpec((_NUM_EMB, _DIM), lambda i: (0, 0)),
    ],
    out_specs=[
        pl.BlockSpec((_TOKENS,), lambda i: (0,)),
        pl.BlockSpec((1, 1), lambda i: (0, 0), memory_space=pltpu.SMEM),
    ],
    out_shape=[
        jax.ShapeDtypeStruct((_TOKENS,), jnp.int32),
        jax.ShapeDtypeStruct((1, 1), jnp.float32),
    ],
    scratch_shapes=[pltpu.VMEM((_NUM_EMB,), jnp.float32),
                    pltpu.VMEM((1, _NUM_EMB), jnp.float32)],
)


@functools.lru_cache(maxsize=None)
def _make_sc_gather(nc, ns):
    nw = nc * ns
    b_per_w = _TOKENS // nw
    mesh = plsc.VectorSubcoreMesh(core_axis_name="c", subcore_axis_name="s",
                                  num_cores=nc, num_subcores=ns)

    @functools.partial(
        pl.kernel,
        mesh=mesh,
        compiler_params=pltpu.CompilerParams(use_tc_tiling_on_sc=False),
        out_type=jax.ShapeDtypeStruct((_B, _F, _DIM), jnp.float32),
        scratch_types=[
            pltpu.VMEM((b_per_w,), jnp.int32),
            pltpu.VMEM((b_per_w, _DIM), jnp.float32),
            pltpu.SemaphoreType.DMA,
        ],
    )
    def _gather(e_hbm, idx_hbm, out_hbm, idx_v, rows_v, sem):
        wid = lax.axis_index("s") * nc + lax.axis_index("c")
        row = (wid * b_per_w) // _F
        col = (wid * b_per_w) % _F
        pltpu.sync_copy(idx_hbm.at[pl.ds(wid * b_per_w, b_per_w)], idx_v)
        pltpu.async_copy(e_hbm.at[idx_v], rows_v, sem).wait()
        pltpu.sync_copy(rows_v, out_hbm.at[row, pl.ds(col, b_per_w), :])

    return _gather


def kernel(z, embedding):
    b, f, dim = z.shape
    idx, loss = _tc_call(z.reshape(-1, dim), embedding)
    info = plsc.get_sparse_core_info()
    zq = _make_sc_gather(1, info.num_subcores)(embedding, idx)
    return zq, idx.reshape(b, f), loss[0, 0]
